# Initial kernel scaffold; baseline (speedup 1.0000x reference)
import jax
import jax.numpy as jnp
from jax.experimental import pallas as pl


def kernel(cosine, label):
    raise NotImplementedError("write your pallas kernel here")


# speedup vs baseline: 1.5039x; 1.5039x over previous
import functools
import math

import jax
import jax.numpy as jnp
from jax.experimental import pallas as pl
from jax.experimental.pallas import tpu as pltpu

_SP = 1.0
_SN = 64.0
_COS_M = math.cos(0.5)
_SIN_M = math.sin(0.5)

_BLOCK_COLS = 2048


def _tc_body(lab_ref, cos_ref, out_ref):
    j = pl.program_id(0)
    x = jnp.clip(cos_ref[...], -1.0, 1.0)
    lab = lab_ref[...]
    col = j * _BLOCK_COLS + jax.lax.broadcasted_iota(jnp.int32, x.shape, 1)
    is_t = col == lab
    t = jnp.max(jnp.where(is_t, x, -2.0), axis=1, keepdims=True)
    adj = (t * _COS_M - jnp.sqrt(jnp.maximum(1.0 - t * t, 0.0)) * _SIN_M) * _SP
    out_ref[...] = jnp.where(is_t, adj, x) * _SN


def kernel(cosine, label):
    b, c = cosine.shape
    grid = pl.cdiv(c, _BLOCK_COLS)
    return pl.pallas_call(
        _tc_body,
        grid=(grid,),
        in_specs=[
            pl.BlockSpec((b, 1), lambda j: (0, 0)),
            pl.BlockSpec((b, _BLOCK_COLS), lambda j: (0, j)),
        ],
        out_specs=pl.BlockSpec((b, _BLOCK_COLS), lambda j: (0, j)),
        out_shape=jax.ShapeDtypeStruct((b, c), cosine.dtype),
        compiler_params=pltpu.CompilerParams(
            dimension_semantics=("arbitrary",),
        ),
    )(label[:, None], cosine)
